# baseline (speedup 1.0000x reference)
import functools

import jax
import jax.numpy as jnp
from jax import lax
from jax.experimental import pallas as pl
from jax.experimental.pallas import tpu as pltpu
from jax.experimental.pallas import tpu_sc as plsc

D = 64
K = 20
NC = 2
NS = 16
NW = NC * NS
LANES = 16

CHUNK_B = 32
ROWS_PER_CHUNK = CHUNK_B * K
IDX_DMA = 128


def _sc_body(batch, vhalf, cw_hbm, xw_hbm, nw_hbm, ie_hbm, oe_hbm,
             pos_hbm, neg_hbm,
             idx_c, idx_x, idx_n, idx_ch, idx_xh, idx_nh,
             cen, ctx, negb, pos_v, neg_v, sem):
    nb = batch // NW
    wid = lax.axis_index("s") * NC + lax.axis_index("c")
    base = wid * nb

    n_idx_rows = nb // IDX_DMA
    for j in range(n_idx_rows):
        pltpu.sync_copy(cw_hbm.at[pl.ds(base + j * IDX_DMA, IDX_DMA)],
                        idx_c.at[j])
        pltpu.sync_copy(xw_hbm.at[pl.ds(base + j * IDX_DMA, IDX_DMA)],
                        idx_x.at[j])
    n_negidx_rows = nb * K // IDX_DMA
    for j in range(n_negidx_rows):
        pltpu.sync_copy(nw_hbm.at[pl.ds(base * K + j * IDX_DMA, IDX_DMA)],
                        idx_n.at[j])

    for j in range(n_idx_rows):
        for t in range(IDX_DMA // LANES):
            s = pl.ds(t * LANES, LANES)
            idx_ch[j, s] = idx_c[j, s] >> 1
            idx_xh[j, s] = idx_x[j, s] >> 1

    def halve_neg(j, _):
        for t in range(IDX_DMA // LANES):
            s = pl.ds(t * LANES, LANES)
            idx_nh[j, s] = idx_n[j, s] >> 1
        return 0

    lax.fori_loop(0, n_negidx_rows, halve_neg, 0)

    iota = lax.iota(jnp.int32, LANES)
    diag = 8 * iota
    n_chunks = nb // CHUNK_B
    groups_per_chunk = CHUNK_B // LANES
    zero = jnp.zeros((LANES,), jnp.float32)

    for c in range(n_chunks):
        cr = c >> 2
        cc = (c & 3) * CHUNK_B
        cps = [
            pltpu.async_copy(
                ie_hbm.at[idx_ch.at[cr, pl.ds(cc, CHUNK_B)]], cen, sem),
            pltpu.async_copy(
                oe_hbm.at[idx_xh.at[cr, pl.ds(cc, CHUNK_B)]], ctx, sem),
        ]
        for j in range(ROWS_PER_CHUNK // IDX_DMA):
            cps.append(pltpu.async_copy(
                oe_hbm.at[idx_nh.at[c * (ROWS_PER_CHUNK // IDX_DMA) + j]],
                negb.at[pl.ds(j * IDX_DMA, IDX_DMA)], sem))
        for cp in cps:
            cp.wait()

        for g in range(groups_per_chunk):
            b0 = c * CHUNK_B + g * LANES
            row_cx = g * LANES + iota
            pb_c = (idx_c[cr, pl.ds(cc + g * LANES, LANES)] & 1) << 6
            pb_x = (idx_x[cr, pl.ds(cc + g * LANES, LANES)] & 1) << 6
            rvecs = []
            pbs = []
            for k in range(K):
                p = (g * LANES + iota) * K + k
                rvecs.append(p)
                q = c * ROWS_PER_CHUNK + p
                pbs.append((plsc.load_gather(
                    idx_n, [q >> 7, q & 127]) & 1) << 6)

            def pass0(d, carry, _pb=pbs, _rv=rvecs, _row=row_cx,
                      _pbc=pb_c, _pbx=pb_x):
                rot = (diag + d) & (D - 1)
                cd = plsc.load_gather(cen, [_row, _pbc + rot])
                xd = plsc.load_gather(ctx, [_row, _pbx + rot])
                accs = tuple(
                    carry[1 + k] + cd * plsc.load_gather(
                        negb, [_rv[k], _pb[k] + rot])
                    for k in range(K // 2))
                return (carry[0] + cd * xd,) + accs

            def pass1(d, carry, _pb=pbs, _rv=rvecs, _row=row_cx,
                      _pbc=pb_c):
                rot = (diag + d) & (D - 1)
                cd = plsc.load_gather(cen, [_row, _pbc + rot])
                return tuple(
                    carry[k] + cd * plsc.load_gather(
                        negb, [_rv[K // 2 + k], _pb[K // 2 + k] + rot])
                    for k in range(K // 2))

            res0 = lax.fori_loop(0, D, pass0, (zero,) * (K // 2 + 1))
            res1 = lax.fori_loop(0, D, pass1, (zero,) * (K // 2))
            pos_v[pl.ds(b0, LANES)] = res0[0]
            for k in range(K // 2):
                neg_v[k, pl.ds(b0, LANES)] = res0[1 + k]
                neg_v[K // 2 + k, pl.ds(b0, LANES)] = res1[k]

    pltpu.sync_copy(pos_v, pos_hbm.at[pl.ds(base, nb)])
    pltpu.sync_copy(neg_v, neg_hbm.at[wid])


def _sc_scores(cw, xw, nw_flat, ie2, oe2, vhalf):
    batch = cw.shape[0]
    nb = batch // NW
    mesh = plsc.VectorSubcoreMesh(core_axis_name="c", subcore_axis_name="s",
                                  num_cores=NC, num_subcores=NS)
    f = pl.kernel(
        functools.partial(_sc_body, batch, vhalf),
        out_type=(jax.ShapeDtypeStruct((batch,), jnp.float32),
                  jax.ShapeDtypeStruct((NW, K, nb), jnp.float32)),
        mesh=mesh,
        compiler_params=pltpu.CompilerParams(
            needs_layout_passes=False, use_tc_tiling_on_sc=True),
        scratch_types=[
            pltpu.VMEM((nb // IDX_DMA, IDX_DMA), jnp.int32),
            pltpu.VMEM((nb // IDX_DMA, IDX_DMA), jnp.int32),
            pltpu.VMEM((nb * K // IDX_DMA, IDX_DMA), jnp.int32),
            pltpu.VMEM((nb // IDX_DMA, IDX_DMA), jnp.int32),
            pltpu.VMEM((nb // IDX_DMA, IDX_DMA), jnp.int32),
            pltpu.VMEM((nb * K // IDX_DMA, IDX_DMA), jnp.int32),
            pltpu.VMEM((CHUNK_B, 2 * D), jnp.float32),
            pltpu.VMEM((CHUNK_B, 2 * D), jnp.float32),
            pltpu.VMEM((ROWS_PER_CHUNK, 2 * D), jnp.float32),
            pltpu.VMEM((nb,), jnp.float32),
            pltpu.VMEM((K, nb), jnp.float32),
            pltpu.SemaphoreType.DMA,
        ],
    )
    return f(cw, xw, nw_flat, ie2, oe2)


PACK_PB = 1024


def _pack_body(a_ref, out_ref):
    t = lax.dot_general(a_ref[...], jnp.eye(D, dtype=jnp.float32),
                        (((0,), (0,)), ((), ())),
                        preferred_element_type=jnp.float32)
    t = t.reshape(PACK_PB, 2, D)
    out_ref[...] = jnp.concatenate([t[:, 0, :], t[:, 1, :]], axis=1)


def _pack(table_t, vocab):
    vh = vocab // 2
    grid = (vh + PACK_PB - 1) // PACK_PB
    return pl.pallas_call(
        _pack_body,
        grid=(grid,),
        in_specs=[pl.BlockSpec((D, 2 * PACK_PB), lambda g: (0, g))],
        out_specs=pl.BlockSpec((PACK_PB, 2 * D), lambda g: (g, 0)),
        out_shape=jax.ShapeDtypeStruct((vh, 2 * D), jnp.float32),
    )(table_t)


def _finish_body(batch, pos_ref, neg_ref, out_ref):
    pos = pos_ref[...]
    neg = neg_ref[...]

    def logsig(x):
        return jnp.minimum(x, 0.0) - jnp.log1p(jnp.exp(-jnp.abs(x)))

    total = jnp.sum(logsig(pos)) + jnp.sum(logsig(-neg))
    out_ref[0, 0] = -total / batch


def _finish(pos2d, neg2d, batch):
    return pl.pallas_call(
        functools.partial(_finish_body, batch),
        out_shape=jax.ShapeDtypeStruct((1, 1), jnp.float32),
        out_specs=pl.BlockSpec(memory_space=pltpu.SMEM),
    )(pos2d, neg2d)


def kernel(center_words, context_words, negative_words,
           input_embeddings, output_embeddings):
    batch = center_words.shape[0]
    vocab = input_embeddings.shape[0]
    cw = center_words.astype(jnp.int32)
    xw = context_words.astype(jnp.int32)
    nw_flat = negative_words.astype(jnp.int32).reshape(-1)
    ie2 = _pack(input_embeddings.T, vocab)
    oe2 = _pack(output_embeddings.T, vocab)
    pos, neg = _sc_scores(cw, xw, nw_flat, ie2, oe2, vocab // 2)
    out = _finish(pos.reshape(NW, batch // NW),
                  neg.reshape(NW * K, batch // NW), batch)
    return out.reshape(())


# speedup vs baseline: 1.3286x; 1.3286x over previous
import functools

import jax
import jax.numpy as jnp
from jax import lax
from jax.experimental import pallas as pl
from jax.experimental.pallas import tpu as pltpu
from jax.experimental.pallas import tpu_sc as plsc

D = 64
K = 20
NC = 2
NS = 16
NW = NC * NS
LANES = 16

CHUNK_B = 32
ROWS_PER_CHUNK = CHUNK_B * K
IDX_DMA = 128


def _sc_body(batch, vhalf, cw_hbm, xw_hbm, nw_hbm, ie_hbm, oe_hbm,
             pos_hbm, neg_hbm,
             idx_c, idx_x, idx_n, idx_ch, idx_xh, idx_nh,
             cen, ctx, negb, pos_v, neg_v, sem):
    nb = batch // NW
    wid = lax.axis_index("s") * NC + lax.axis_index("c")
    base = wid * nb

    n_idx_rows = nb // IDX_DMA
    for j in range(n_idx_rows):
        pltpu.sync_copy(cw_hbm.at[pl.ds(base + j * IDX_DMA, IDX_DMA)],
                        idx_c.at[j])
        pltpu.sync_copy(xw_hbm.at[pl.ds(base + j * IDX_DMA, IDX_DMA)],
                        idx_x.at[j])
    n_negidx_rows = nb * K // IDX_DMA
    for j in range(n_negidx_rows):
        pltpu.sync_copy(nw_hbm.at[pl.ds(base * K + j * IDX_DMA, IDX_DMA)],
                        idx_n.at[j])

    for j in range(n_idx_rows):
        for t in range(IDX_DMA // LANES):
            s = pl.ds(t * LANES, LANES)
            idx_ch[j, s] = idx_c[j, s] >> 1
            idx_xh[j, s] = idx_x[j, s] >> 1

    def halve_neg(j, _):
        for t in range(IDX_DMA // LANES):
            s = pl.ds(t * LANES, LANES)
            idx_nh[j, s] = idx_n[j, s] >> 1
        return 0

    lax.fori_loop(0, n_negidx_rows, halve_neg, 0)

    iota = lax.iota(jnp.int32, LANES)
    diag = 8 * iota
    n_chunks = nb // CHUNK_B
    groups_per_chunk = CHUNK_B // LANES
    zero = jnp.zeros((LANES,), jnp.float32)

    for c in range(n_chunks):
        cr = c >> 2
        cc = (c & 3) * CHUNK_B
        cps = [
            pltpu.async_copy(
                ie_hbm.at[idx_ch.at[cr, pl.ds(cc, CHUNK_B)]], cen, sem),
            pltpu.async_copy(
                oe_hbm.at[idx_xh.at[cr, pl.ds(cc, CHUNK_B)]], ctx, sem),
        ]
        for j in range(ROWS_PER_CHUNK // IDX_DMA):
            cps.append(pltpu.async_copy(
                oe_hbm.at[idx_nh.at[c * (ROWS_PER_CHUNK // IDX_DMA) + j]],
                negb.at[pl.ds(j * IDX_DMA, IDX_DMA)], sem))
        for cp in cps:
            cp.wait()

        for g in range(groups_per_chunk):
            b0 = c * CHUNK_B + g * LANES
            row_cx = g * LANES + iota
            pb_c = (idx_c[cr, pl.ds(cc + g * LANES, LANES)] & 1) << 6
            pb_x = (idx_x[cr, pl.ds(cc + g * LANES, LANES)] & 1) << 6
            rvecs = []
            pbs = []
            for k in range(K):
                p = (g * LANES + iota) * K + k
                rvecs.append(p)
                q = c * ROWS_PER_CHUNK + p
                pbs.append((plsc.load_gather(
                    idx_n, [q >> 7, q & 127]) & 1) << 6)

            def pass0(d, carry, _pb=pbs, _rv=rvecs, _row=row_cx,
                      _pbc=pb_c, _pbx=pb_x):
                rot = (diag + d) & (D - 1)
                cd = plsc.load_gather(cen, [_row, _pbc + rot])
                xd = plsc.load_gather(ctx, [_row, _pbx + rot])
                accs = tuple(
                    carry[1 + k] + cd * plsc.load_gather(
                        negb, [_rv[k], _pb[k] + rot])
                    for k in range(K // 2))
                return (carry[0] + cd * xd,) + accs

            def pass1(d, carry, _pb=pbs, _rv=rvecs, _row=row_cx,
                      _pbc=pb_c):
                rot = (diag + d) & (D - 1)
                cd = plsc.load_gather(cen, [_row, _pbc + rot])
                return tuple(
                    carry[k] + cd * plsc.load_gather(
                        negb, [_rv[K // 2 + k], _pb[K // 2 + k] + rot])
                    for k in range(K // 2))

            res0 = lax.fori_loop(0, D, pass0, (zero,) * (K // 2 + 1))
            res1 = lax.fori_loop(0, D, pass1, (zero,) * (K // 2))
            pos_v[pl.ds(b0, LANES)] = res0[0]
            for k in range(K // 2):
                neg_v[k, pl.ds(b0, LANES)] = res0[1 + k]
                neg_v[K // 2 + k, pl.ds(b0, LANES)] = res1[k]

    pltpu.sync_copy(pos_v, pos_hbm.at[pl.ds(base, nb)])
    pltpu.sync_copy(neg_v, neg_hbm.at[wid])


def _sc_scores(cw, xw, nw_flat, ie2, oe2, vhalf):
    batch = cw.shape[0]
    nb = batch // NW
    mesh = plsc.VectorSubcoreMesh(core_axis_name="c", subcore_axis_name="s",
                                  num_cores=NC, num_subcores=NS)
    f = pl.kernel(
        functools.partial(_sc_body, batch, vhalf),
        out_type=(jax.ShapeDtypeStruct((batch,), jnp.float32),
                  jax.ShapeDtypeStruct((NW, K, nb), jnp.float32)),
        mesh=mesh,
        compiler_params=pltpu.CompilerParams(
            needs_layout_passes=False, use_tc_tiling_on_sc=True),
        scratch_types=[
            pltpu.VMEM((nb // IDX_DMA, IDX_DMA), jnp.int32),
            pltpu.VMEM((nb // IDX_DMA, IDX_DMA), jnp.int32),
            pltpu.VMEM((nb * K // IDX_DMA, IDX_DMA), jnp.int32),
            pltpu.VMEM((nb // IDX_DMA, IDX_DMA), jnp.int32),
            pltpu.VMEM((nb // IDX_DMA, IDX_DMA), jnp.int32),
            pltpu.VMEM((nb * K // IDX_DMA, IDX_DMA), jnp.int32),
            pltpu.VMEM((CHUNK_B, 2 * D), jnp.float32),
            pltpu.VMEM((CHUNK_B, 2 * D), jnp.float32),
            pltpu.VMEM((ROWS_PER_CHUNK, 2 * D), jnp.float32),
            pltpu.VMEM((nb,), jnp.float32),
            pltpu.VMEM((K, nb), jnp.float32),
            pltpu.SemaphoreType.DMA,
        ],
    )
    return f(cw, xw, nw_flat, ie2, oe2)


PACK_PB = 4096


def _pack_body(a_ref, out_ref):
    t = a_ref[...].T.reshape(PACK_PB, 2, D)
    out_ref[...] = jnp.concatenate([t[:, 0, :], t[:, 1, :]], axis=1)


def _pack(table_t, vocab):
    vh = vocab // 2
    grid = (vh + PACK_PB - 1) // PACK_PB
    return pl.pallas_call(
        _pack_body,
        grid=(grid,),
        in_specs=[pl.BlockSpec((D, 2 * PACK_PB), lambda g: (0, g))],
        out_specs=pl.BlockSpec((PACK_PB, 2 * D), lambda g: (g, 0)),
        out_shape=jax.ShapeDtypeStruct((vh, 2 * D), jnp.float32),
    )(table_t)


def _finish_body(batch, pos_ref, neg_ref, out_ref):
    pos = pos_ref[...]
    neg = neg_ref[...]

    def logsig(x):
        return jnp.minimum(x, 0.0) - jnp.log1p(jnp.exp(-jnp.abs(x)))

    total = jnp.sum(logsig(pos)) + jnp.sum(logsig(-neg))
    out_ref[0, 0] = -total / batch


def _finish(pos2d, neg2d, batch):
    return pl.pallas_call(
        functools.partial(_finish_body, batch),
        out_shape=jax.ShapeDtypeStruct((1, 1), jnp.float32),
        out_specs=pl.BlockSpec(memory_space=pltpu.SMEM),
    )(pos2d, neg2d)


def kernel(center_words, context_words, negative_words,
           input_embeddings, output_embeddings):
    batch = center_words.shape[0]
    vocab = input_embeddings.shape[0]
    cw = center_words.astype(jnp.int32)
    xw = context_words.astype(jnp.int32)
    nw_flat = negative_words.astype(jnp.int32).reshape(-1)
    ie2 = _pack(input_embeddings.T, vocab)
    oe2 = _pack(output_embeddings.T, vocab)
    pos, neg = _sc_scores(cw, xw, nw_flat, ie2, oe2, vocab // 2)
    out = _finish(pos.reshape(NW, batch // NW),
                  neg.reshape(NW * K, batch // NW), batch)
    return out.reshape(())


# speedup vs baseline: 1.3372x; 1.0065x over previous
import functools

import jax
import jax.numpy as jnp
from jax import lax
from jax.experimental import pallas as pl
from jax.experimental.pallas import tpu as pltpu
from jax.experimental.pallas import tpu_sc as plsc

D = 64
K = 20
NC = 2
NS = 16
NW = NC * NS
LANES = 16

CHUNK_B = 32
ROWS_PER_CHUNK = CHUNK_B * K
IDX_DMA = 128


def _sc_body(batch, vhalf, cw_hbm, xw_hbm, nw_hbm, ie_hbm, oe_hbm,
             pos_hbm, neg_hbm,
             idx_c, idx_x, idx_n, idx_ch, idx_xh, idx_nh,
             cen, ctx, negb, pos_v, neg_v, sem):
    nb = batch // NW
    wid = lax.axis_index("s") * NC + lax.axis_index("c")
    base = wid * nb

    n_idx_rows = nb // IDX_DMA
    for j in range(n_idx_rows):
        pltpu.sync_copy(cw_hbm.at[pl.ds(base + j * IDX_DMA, IDX_DMA)],
                        idx_c.at[j])
        pltpu.sync_copy(xw_hbm.at[pl.ds(base + j * IDX_DMA, IDX_DMA)],
                        idx_x.at[j])
    n_negidx_rows = nb * K // IDX_DMA
    for j in range(n_negidx_rows):
        pltpu.sync_copy(nw_hbm.at[pl.ds(base * K + j * IDX_DMA, IDX_DMA)],
                        idx_n.at[j])

    for j in range(n_idx_rows):
        for t in range(IDX_DMA // LANES):
            s = pl.ds(t * LANES, LANES)
            idx_ch[j, s] = idx_c[j, s] >> 1
            idx_xh[j, s] = idx_x[j, s] >> 1

    def halve_neg(j, _):
        for t in range(IDX_DMA // LANES):
            s = pl.ds(t * LANES, LANES)
            idx_nh[j, s] = idx_n[j, s] >> 1
        return 0

    lax.fori_loop(0, n_negidx_rows, halve_neg, 0)

    iota = lax.iota(jnp.int32, LANES)
    diag = 8 * iota
    n_chunks = nb // CHUNK_B
    groups_per_chunk = CHUNK_B // LANES
    zero = jnp.zeros((LANES,), jnp.float32)

    for c in range(n_chunks):
        cr = c >> 2
        cc = (c & 3) * CHUNK_B
        cps = [
            pltpu.async_copy(
                ie_hbm.at[idx_ch.at[cr, pl.ds(cc, CHUNK_B)]], cen, sem),
            pltpu.async_copy(
                oe_hbm.at[idx_xh.at[cr, pl.ds(cc, CHUNK_B)]], ctx, sem),
        ]
        for j in range(ROWS_PER_CHUNK // IDX_DMA):
            cps.append(pltpu.async_copy(
                oe_hbm.at[idx_nh.at[c * (ROWS_PER_CHUNK // IDX_DMA) + j]],
                negb.at[pl.ds(j * IDX_DMA, IDX_DMA)], sem))
        for cp in cps:
            cp.wait()

        for g in range(groups_per_chunk):
            b0 = c * CHUNK_B + g * LANES
            row_cx = g * LANES + iota
            pb_c = (idx_c[cr, pl.ds(cc + g * LANES, LANES)] & 1) << 6
            pb_x = (idx_x[cr, pl.ds(cc + g * LANES, LANES)] & 1) << 6
            rvecs = []
            pbs = []
            for k in range(K):
                p = (g * LANES + iota) * K + k
                rvecs.append(p)
                q = c * ROWS_PER_CHUNK + p
                pbs.append((plsc.load_gather(
                    idx_n, [q >> 7, q & 127]) & 1) << 6)

            def pass0(d, carry, _pb=pbs, _rv=rvecs, _row=row_cx,
                      _pbc=pb_c, _pbx=pb_x):
                rot = (diag + d) & (D - 1)
                cd = plsc.load_gather(cen, [_row, _pbc + rot])
                xd = plsc.load_gather(ctx, [_row, _pbx + rot])
                accs = tuple(
                    carry[1 + k] + cd * plsc.load_gather(
                        negb, [_rv[k], _pb[k] + rot])
                    for k in range(K // 2))
                return (carry[0] + cd * xd,) + accs

            def pass1(d, carry, _pb=pbs, _rv=rvecs, _row=row_cx,
                      _pbc=pb_c):
                rot = (diag + d) & (D - 1)
                cd = plsc.load_gather(cen, [_row, _pbc + rot])
                return tuple(
                    carry[k] + cd * plsc.load_gather(
                        negb, [_rv[K // 2 + k], _pb[K // 2 + k] + rot])
                    for k in range(K // 2))

            res0 = lax.fori_loop(0, D, pass0, (zero,) * (K // 2 + 1))
            res1 = lax.fori_loop(0, D, pass1, (zero,) * (K // 2))
            pos_v[pl.ds(b0, LANES)] = res0[0]
            for k in range(K // 2):
                neg_v[k, pl.ds(b0, LANES)] = res0[1 + k]
                neg_v[K // 2 + k, pl.ds(b0, LANES)] = res1[k]

    pltpu.sync_copy(pos_v, pos_hbm.at[pl.ds(base, nb)])
    pltpu.sync_copy(neg_v, neg_hbm.at[wid])


def _sc_scores(cw, xw, nw_flat, ie2, oe2, vhalf):
    batch = cw.shape[0]
    nb = batch // NW
    mesh = plsc.VectorSubcoreMesh(core_axis_name="c", subcore_axis_name="s",
                                  num_cores=NC, num_subcores=NS)
    f = pl.kernel(
        functools.partial(_sc_body, batch, vhalf),
        out_type=(jax.ShapeDtypeStruct((batch,), jnp.float32),
                  jax.ShapeDtypeStruct((NW, K, nb), jnp.float32)),
        mesh=mesh,
        compiler_params=pltpu.CompilerParams(
            needs_layout_passes=False, use_tc_tiling_on_sc=True),
        scratch_types=[
            pltpu.VMEM((nb // IDX_DMA, IDX_DMA), jnp.int32),
            pltpu.VMEM((nb // IDX_DMA, IDX_DMA), jnp.int32),
            pltpu.VMEM((nb * K // IDX_DMA, IDX_DMA), jnp.int32),
            pltpu.VMEM((nb // IDX_DMA, IDX_DMA), jnp.int32),
            pltpu.VMEM((nb // IDX_DMA, IDX_DMA), jnp.int32),
            pltpu.VMEM((nb * K // IDX_DMA, IDX_DMA), jnp.int32),
            pltpu.VMEM((CHUNK_B, 2 * D), jnp.float32),
            pltpu.VMEM((CHUNK_B, 2 * D), jnp.float32),
            pltpu.VMEM((ROWS_PER_CHUNK, 2 * D), jnp.float32),
            pltpu.VMEM((nb,), jnp.float32),
            pltpu.VMEM((K, nb), jnp.float32),
            pltpu.SemaphoreType.DMA,
        ],
    )
    return f(cw, xw, nw_flat, ie2, oe2)


PACK_PB = 8192


def _pack_body(a_ref, out_ref):
    t = a_ref[...].T.reshape(PACK_PB, 2, D)
    out_ref[...] = jnp.concatenate([t[:, 0, :], t[:, 1, :]], axis=1)


def _pack(table_t, vocab):
    vh = vocab // 2
    grid = (vh + PACK_PB - 1) // PACK_PB
    return pl.pallas_call(
        _pack_body,
        grid=(grid,),
        in_specs=[pl.BlockSpec((D, 2 * PACK_PB), lambda g: (0, g))],
        out_specs=pl.BlockSpec((PACK_PB, 2 * D), lambda g: (g, 0)),
        out_shape=jax.ShapeDtypeStruct((vh, 2 * D), jnp.float32),
    )(table_t)


def _finish_body(batch, pos_ref, neg_ref, out_ref):
    pos = pos_ref[...]
    neg = neg_ref[...]

    def logsig(x):
        return jnp.minimum(x, 0.0) - jnp.log1p(jnp.exp(-jnp.abs(x)))

    total = jnp.sum(logsig(pos)) + jnp.sum(logsig(-neg))
    out_ref[0, 0] = -total / batch


def _finish(pos2d, neg2d, batch):
    return pl.pallas_call(
        functools.partial(_finish_body, batch),
        out_shape=jax.ShapeDtypeStruct((1, 1), jnp.float32),
        out_specs=pl.BlockSpec(memory_space=pltpu.SMEM),
    )(pos2d, neg2d)


def kernel(center_words, context_words, negative_words,
           input_embeddings, output_embeddings):
    batch = center_words.shape[0]
    vocab = input_embeddings.shape[0]
    cw = center_words.astype(jnp.int32)
    xw = context_words.astype(jnp.int32)
    nw_flat = negative_words.astype(jnp.int32).reshape(-1)
    ie2 = _pack(input_embeddings.T, vocab)
    oe2 = _pack(output_embeddings.T, vocab)
    pos, neg = _sc_scores(cw, xw, nw_flat, ie2, oe2, vocab // 2)
    out = _finish(pos.reshape(NW, batch // NW),
                  neg.reshape(NW * K, batch // NW), batch)
    return out.reshape(())
